# baseline (speedup 1.0000x reference)
import jax
import jax.numpy as jnp
from jax.experimental import pallas as pl
from jax.experimental.pallas import tpu as pltpu

_NUM_CLASSES = 256
_C = _NUM_CLASSES + 1
_TOPK = 512
_N = 131072
_BR = 512
_INT_MIN = -2147483648


def _stats_kernel(s_ref, lab_ref, pos_ref, neg_ref, con_ref):
    s = s_ref[...]
    lab = lab_ref[...]
    col = jax.lax.broadcasted_iota(jnp.int32, s.shape, 1)
    neg_inf = jnp.float32(-jnp.inf)

    m_all = jnp.max(s, axis=1)
    m_drop = jnp.max(jnp.where(col == _NUM_CLASSES - 2, neg_inf, s), axis=1)
    metric = -m_drop

    e = jnp.exp(s - m_all[:, None])
    se = jnp.sum(e, axis=1)
    onehot = col == lab[:, None]
    el = jnp.sum(jnp.where(onehot, e, 0.0), axis=1)
    se_excl = jnp.maximum(se - el, jnp.float32(1e-30))
    zm = m_all + jnp.log(se_excl)
    gt = el / se
    val = gt * (1.0 - gt)

    s255 = jnp.sum(jnp.where(col == _C - 2, s, 0.0), axis=1)
    s256 = jnp.sum(jnp.where(col == _C - 1, s, 0.0), axis=1)
    fg = lab != _NUM_CLASSES - 1
    x = jnp.where(fg, s255, s256)
    contrib = val * (zm - x)

    b = jax.lax.bitcast_convert_type(metric, jnp.int32)
    key = jnp.where(b < 0, b ^ jnp.int32(0x7FFFFFFF), b)
    pos_ref[...] = jnp.where(fg, key, jnp.int32(_INT_MIN))
    neg_ref[...] = jnp.where(fg, jnp.int32(_INT_MIN), key)
    con_ref[...] = contrib


def _select_kernel(posk_ref, negk_ref, con_ref, out_ref):
    posk = posk_ref[...]
    negk = negk_ref[...]
    con = con_ref[...]
    num_bg = jnp.sum((negk != jnp.int32(_INT_MIN)).astype(jnp.int32))
    num_fg = _N - num_bg
    topk = jnp.minimum(jnp.minimum(num_fg, num_bg), jnp.int32(_TOPK))

    r0 = jax.lax.broadcasted_iota(jnp.int32, posk.shape, 0)
    r1 = jax.lax.broadcasted_iota(jnp.int32, posk.shape, 1)
    idx = r0 * posk.shape[1] + r1

    def group_sum(keys):
        def tbody(i, u):
            cand_u = u | (jnp.int32(1) << (31 - i))
            cnt = jnp.sum((keys >= (cand_u ^ jnp.int32(_INT_MIN))).astype(jnp.int32))
            return jnp.where(cnt >= topk, cand_u, u)
        u = jax.lax.fori_loop(0, 32, tbody, jnp.int32(0))
        t = u ^ jnp.int32(_INT_MIN)
        cnt_gt = jnp.sum((keys > t).astype(jnp.int32))
        need = topk - cnt_gt
        tie = keys == t
        def xbody(i, xm):
            cand = xm | (jnp.int32(1) << (17 - i))
            g = jnp.sum((tie & (idx < cand)).astype(jnp.int32))
            return jnp.where(g < need, cand, xm)
        xm = jax.lax.fori_loop(0, 18, xbody, jnp.int32(0))
        xmin = jnp.where(need > 0, xm + 1, jnp.int32(0))
        sel = (keys > t) | (tie & (idx < xmin))
        return jnp.sum(jnp.where(sel, con, 0.0))

    total = group_sum(posk) + group_sum(negk)
    denom = jnp.maximum(2.0 * topk.astype(jnp.float32), jnp.float32(1.0))
    loss = jnp.where(topk > 0, total / denom, jnp.float32(0.0))
    out_ref[...] = jnp.broadcast_to(loss, (1, 1))


def kernel(scores, labels):
    grid = _N // _BR
    posk, negk, con = pl.pallas_call(
        _stats_kernel,
        grid=(grid,),
        in_specs=[
            pl.BlockSpec((_BR, _C), lambda i: (i, 0)),
            pl.BlockSpec((_BR,), lambda i: (i,)),
        ],
        out_specs=[pl.BlockSpec((_BR,), lambda i: (i,))] * 3,
        out_shape=[
            jax.ShapeDtypeStruct((_N,), jnp.int32),
            jax.ShapeDtypeStruct((_N,), jnp.int32),
            jax.ShapeDtypeStruct((_N,), jnp.float32),
        ],
    )(scores, labels.astype(jnp.int32))

    R = _N // 512
    loss = pl.pallas_call(
        _select_kernel,
        out_shape=jax.ShapeDtypeStruct((1, 1), jnp.float32),
    )(posk.reshape(R, 512), negk.reshape(R, 512), con.reshape(R, 512))
    return loss[0, 0]


# speedup vs baseline: 1.4011x; 1.4011x over previous
import jax
import jax.numpy as jnp
from jax.experimental import pallas as pl
from jax.experimental.pallas import tpu as pltpu

_NUM_CLASSES = 256
_C = _NUM_CLASSES + 1
_TOPK = 512
_N = 131072
_BR = 4096
_INT_MIN = -2147483648


def _stats_kernel(s_ref, lab_ref, pos_ref, neg_ref, con_ref):
    s = s_ref[...]
    lab = lab_ref[...]
    col = jax.lax.broadcasted_iota(jnp.int32, s.shape, 1)
    neg_inf = jnp.float32(-jnp.inf)

    m_drop = jnp.max(jnp.where(col == _NUM_CLASSES - 2, neg_inf, s), axis=1,
                     keepdims=True)
    metric = -m_drop

    m3 = (jax.lax.broadcasted_iota(jnp.int32, (_C, 3), 0)
          == _C - 3 + jax.lax.broadcasted_iota(jnp.int32, (_C, 3), 1)
          ).astype(jnp.float32)
    s3 = jax.lax.dot_general(s, m3, (((1,), (0,)), ((), ())),
                             preferred_element_type=jnp.float32)
    s254 = s3[:, 0:1]
    s255 = s3[:, 1:2]
    s256 = s3[:, 2:3]
    m_all = jnp.maximum(m_drop, s254)

    onehot = col == lab
    ones_c = jnp.ones((_C, 1), dtype=jnp.float32)
    sl = jax.lax.dot_general(jnp.where(onehot, s, 0.0), ones_c,
                             (((1,), (0,)), ((), ())),
                             preferred_element_type=jnp.float32)

    e = jnp.exp(s - m_all)
    se = jax.lax.dot_general(e, ones_c, (((1,), (0,)), ((), ())),
                             preferred_element_type=jnp.float32)
    el = jnp.exp(sl - m_all)
    se_excl = jnp.maximum(se - el, jnp.float32(1e-30))
    zm = m_all + jnp.log(se_excl)
    gt = el / se
    val = gt * (1.0 - gt)

    fg = lab != _NUM_CLASSES - 1
    x = jnp.where(fg, s255, s256)
    contrib = val * (zm - x)

    b = jax.lax.bitcast_convert_type(metric, jnp.int32)
    key = jnp.where(b < 0, b ^ jnp.int32(0x7FFFFFFF), b)
    rows = _BR // 512
    pos_ref[...] = jnp.where(fg, key, jnp.int32(_INT_MIN)).reshape(rows, 512)
    neg_ref[...] = jnp.where(fg, jnp.int32(_INT_MIN), key).reshape(rows, 512)
    con_ref[...] = contrib.reshape(rows, 512)


def _select_kernel(posk_ref, negk_ref, con_ref, out_ref):
    posk = posk_ref[...]
    negk = negk_ref[...]
    con = con_ref[...]
    num_bg = jnp.sum((negk != jnp.int32(_INT_MIN)).astype(jnp.int32))
    num_fg = _N - num_bg
    topk = jnp.minimum(jnp.minimum(num_fg, num_bg), jnp.int32(_TOPK))

    r0 = jax.lax.broadcasted_iota(jnp.int32, posk.shape, 0)
    r1 = jax.lax.broadcasted_iota(jnp.int32, posk.shape, 1)
    idx = r0 * posk.shape[1] + r1

    def group_sum(keys):
        def tbody(i, u):
            cand_u = u | (jnp.int32(1) << (31 - i))
            cnt = jnp.sum((keys >= (cand_u ^ jnp.int32(_INT_MIN))).astype(jnp.int32))
            return jnp.where(cnt >= topk, cand_u, u)
        u = jax.lax.fori_loop(0, 32, tbody, jnp.int32(0))
        t = u ^ jnp.int32(_INT_MIN)
        cnt_gt = jnp.sum((keys > t).astype(jnp.int32))
        need = topk - cnt_gt
        tie = keys == t
        def xbody(i, xm):
            cand = xm | (jnp.int32(1) << (17 - i))
            g = jnp.sum((tie & (idx < cand)).astype(jnp.int32))
            return jnp.where(g < need, cand, xm)
        xm = jax.lax.fori_loop(0, 18, xbody, jnp.int32(0))
        xmin = jnp.where(need > 0, xm + 1, jnp.int32(0))
        sel = (keys > t) | (tie & (idx < xmin))
        return jnp.sum(jnp.where(sel, con, 0.0))

    total = group_sum(posk) + group_sum(negk)
    denom = jnp.maximum(2.0 * topk.astype(jnp.float32), jnp.float32(1.0))
    loss = jnp.where(topk > 0, total / denom, jnp.float32(0.0))
    out_ref[...] = jnp.broadcast_to(loss, (1, 1))


def kernel(scores, labels):
    grid = _N // _BR
    rows = _BR // 512
    R = _N // 512
    posk, negk, con = pl.pallas_call(
        _stats_kernel,
        grid=(grid,),
        in_specs=[
            pl.BlockSpec((_BR, _C), lambda i: (i, 0)),
            pl.BlockSpec((_BR, 1), lambda i: (i, 0)),
        ],
        out_specs=[pl.BlockSpec((rows, 512), lambda i: (i, 0))] * 3,
        out_shape=[
            jax.ShapeDtypeStruct((R, 512), jnp.int32),
            jax.ShapeDtypeStruct((R, 512), jnp.int32),
            jax.ShapeDtypeStruct((R, 512), jnp.float32),
        ],
    )(scores, labels.astype(jnp.int32).reshape(_N, 1))

    loss = pl.pallas_call(
        _select_kernel,
        out_shape=jax.ShapeDtypeStruct((1, 1), jnp.float32),
    )(posk, negk, con)
    return loss[0, 0]
